# Initial kernel scaffold; baseline (speedup 1.0000x reference)
import functools

import jax
import jax.numpy as jnp
from jax import lax
from jax.experimental import pallas as pl
from jax.experimental.pallas import tpu as pltpu
from jax.experimental.pallas import tpu_sc as plsc

N = 10000
E = 320000
D = 128
H = 128
N_PRED = 8
N_GRAPHS = 64

NC = 2
NS = 16
NW = NC * NS
K = 80
EPW = E // NW
CH = EPW // K
N2 = 10240
SLAB = N2 // NS
RPS = N // NS
ZR = 25

_mesh = plsc.VectorSubcoreMesh(core_axis_name="c", subcore_axis_name="s")


@functools.partial(
    pl.kernel,
    out_type=jax.ShapeDtypeStruct((NC, N2), jnp.float32),
    mesh=_mesh,
    scratch_types=[
        pltpu.VMEM_SHARED((N2,), jnp.float32),
        pltpu.VMEM((CH, K), jnp.int32),
        pltpu.VMEM((K,), jnp.float32),
        pltpu.VMEM((SLAB,), jnp.float32),
    ],
)
def _deg_call(dst_hbm, out_hbm, dacc, didx, ones, zb):
    cid = lax.axis_index("c")
    sid = lax.axis_index("s")
    w = sid * NC + cid
    for j in range(K // 16):
        ones[pl.ds(j * 16, 16)] = jnp.ones((16,), jnp.float32)
    for j in range(SLAB // 16):
        zb[pl.ds(j * 16, 16)] = jnp.zeros((16,), jnp.float32)
    pltpu.sync_copy(zb, dacc.at[pl.ds(sid * SLAB, SLAB)])
    pltpu.sync_copy(dst_hbm.at[w], didx)
    plsc.subcore_barrier()

    def body(c, _):
        pltpu.sync_copy(ones, dacc.at[didx.at[c]], add=True)
        return 0

    lax.fori_loop(0, CH, body, 0)
    plsc.subcore_barrier()
    pltpu.sync_copy(dacc.at[pl.ds(sid * SLAB, SLAB)],
                    out_hbm.at[cid, pl.ds(sid * SLAB, SLAB)])


@functools.partial(
    pl.kernel,
    out_type=jax.ShapeDtypeStruct((NC, N, H), jnp.float32),
    mesh=_mesh,
    scratch_types=[
        pltpu.VMEM_SHARED((N, H), jnp.float32),
        pltpu.VMEM((CH, K), jnp.int32),
        pltpu.VMEM((CH, K), jnp.int32),
        pltpu.VMEM((K, H), jnp.float32),
        pltpu.VMEM((K, H), jnp.float32),
        pltpu.VMEM((ZR, H), jnp.float32),
        pltpu.SemaphoreType.DMA,
        pltpu.SemaphoreType.DMA,
    ],
)
def _agg_call(y_hbm, src_hbm, dst_hbm, out_hbm,
              acc, sidx, didx, rows0, rows1, zbuf, sem0, sem1):
    cid = lax.axis_index("c")
    sid = lax.axis_index("s")
    w = sid * NC + cid
    for r in range(ZR):
        for j in range(H // 16):
            zbuf[r, pl.ds(j * 16, 16)] = jnp.zeros((16,), jnp.float32)

    def zslab(t, _):
        pltpu.sync_copy(zbuf, acc.at[pl.ds(sid * RPS + t * ZR, ZR)])
        return 0

    lax.fori_loop(0, RPS // ZR, zslab, 0)
    pltpu.sync_copy(src_hbm.at[w], sidx)
    pltpu.sync_copy(dst_hbm.at[w], didx)
    plsc.subcore_barrier()

    pltpu.async_copy(y_hbm.at[sidx.at[0]], rows0, sem0)

    def body(i, _):
        c0 = 2 * i
        pltpu.async_copy(y_hbm.at[sidx.at[c0 + 1]], rows1, sem1)
        pltpu.make_async_copy(y_hbm.at[sidx.at[c0]], rows0, sem0).wait()
        pltpu.sync_copy(rows0, acc.at[didx.at[c0]], add=True)
        pltpu.async_copy(y_hbm.at[sidx.at[c0 + 2]], rows0, sem0)
        pltpu.make_async_copy(y_hbm.at[sidx.at[c0 + 1]], rows1, sem1).wait()
        pltpu.sync_copy(rows1, acc.at[didx.at[c0 + 1]], add=True)
        return 0

    lax.fori_loop(0, (CH - 1) // 2, body, 0)
    pltpu.make_async_copy(y_hbm.at[sidx.at[CH - 1]], rows0, sem0).wait()
    pltpu.sync_copy(rows0, acc.at[didx.at[CH - 1]], add=True)
    plsc.subcore_barrier()
    pltpu.sync_copy(acc.at[pl.ds(sid * RPS, RPS)],
                    out_hbm.at[cid, pl.ds(sid * RPS, RPS)])


def _tc0_body(degs_ref, x_ref, w_ref, y_ref, dinv_ref):
    deg = degs_ref[0] + degs_ref[1] + 1.0
    dinv = lax.rsqrt(deg)
    dinv_ref[...] = dinv
    xw = jnp.dot(x_ref[...], w_ref[...], preferred_element_type=jnp.float32)
    y_ref[...] = xw * dinv


def _tc_first(degs, x, W0):
    return pl.pallas_call(
        _tc0_body,
        out_shape=[jax.ShapeDtypeStruct((N, H), jnp.float32),
                   jax.ShapeDtypeStruct((N, 1), jnp.float32)],
    )(degs, x, W0)


def _tcmid_body(s_ref, y_ref, dinv_ref, b_ref, w_ref, ynext_ref):
    h = s_ref[0] + s_ref[1] + y_ref[...]
    h = jnp.maximum(h * dinv_ref[...] + b_ref[...], 0.0)
    hw = jnp.dot(h, w_ref[...], preferred_element_type=jnp.float32)
    ynext_ref[...] = hw * dinv_ref[...]


def _tc_mid(s, y, dinv, b, Wn):
    return pl.pallas_call(
        _tcmid_body,
        out_shape=jax.ShapeDtypeStruct((N, H), jnp.float32),
    )(s, y, dinv, b, Wn)


def _tcfin_body(s_ref, y_ref, dinv_ref, b_ref, batch_ref, wl_ref, bl_ref,
                out_ref):
    h = s_ref[0] + s_ref[1] + y_ref[...]
    h = jnp.maximum(h * dinv_ref[...] + b_ref[...], 0.0)
    g = lax.broadcasted_iota(jnp.int32, (N, N_GRAPHS), 1)
    oh = (batch_ref[...] == g).astype(jnp.float32)
    sums = lax.dot_general(oh, h, (((0,), (0,)), ((), ())),
                           preferred_element_type=jnp.float32)
    counts = jnp.sum(oh, axis=0)
    pooled = sums / jnp.maximum(counts, 1.0)[:, None]
    out_ref[...] = jnp.dot(pooled, wl_ref[...],
                           preferred_element_type=jnp.float32) + bl_ref[...]


def _tc_final(s, y, dinv, b, batch, W_lin, b_lin):
    return pl.pallas_call(
        _tcfin_body,
        out_shape=jax.ShapeDtypeStruct((N_GRAPHS, N_PRED), jnp.float32),
    )(s, y, dinv, b, batch, W_lin, b_lin)


def kernel(x, edge_index, batch, W0, b0, W1, b1, W2, b2, W_lin, b_lin):
    src = edge_index[0].reshape(NW, CH, K)
    dst = edge_index[1].reshape(NW, CH, K)
    deg2 = _deg_call(dst)
    degs = deg2[:, :N].reshape(NC, N, 1)
    y0, dinv = _tc_first(degs, x, W0)
    s0 = _agg_call(y0, src, dst)
    y1 = _tc_mid(s0, y0, dinv, b0.reshape(1, H), W1)
    s1 = _agg_call(y1, src, dst)
    y2 = _tc_mid(s1, y1, dinv, b1.reshape(1, H), W2)
    s2 = _agg_call(y2, src, dst)
    return _tc_final(s2, y2, dinv, b2.reshape(1, H), batch.reshape(N, 1),
                     W_lin, b_lin.reshape(1, N_PRED))


# speedup vs baseline: 9.2302x; 9.2302x over previous
import functools

import jax
import jax.numpy as jnp
from jax import lax
from jax.experimental import pallas as pl
from jax.experimental.pallas import tpu as pltpu
from jax.experimental.pallas import tpu_sc as plsc

N = 10000
E = 320000
D = 128
H = 128
N_PRED = 8
N_GRAPHS = 64

NC = 2
NS = 16
NW = NC * NS
EPW = E // NW
K = 80
CH = EPW // K
KA = 128
BCH = 10
NB = 8
EPWP = NB * BCH * KA
N2 = 10240
SLAB = N2 // NS
NA = 10112
RPS = NA // NS
ZR = 8

_mesh = plsc.VectorSubcoreMesh(core_axis_name="c", subcore_axis_name="s",
                               num_cores=NC, num_subcores=NS)


@functools.partial(
    pl.kernel,
    out_type=jax.ShapeDtypeStruct((NC * N2,), jnp.float32),
    mesh=_mesh,
    scratch_types=[
        pltpu.VMEM_SHARED((N2,), jnp.float32),
        pltpu.VMEM((CH, K), jnp.int32),
        pltpu.VMEM((K,), jnp.float32),
        pltpu.VMEM((SLAB,), jnp.float32),
    ],
)
def _deg_call(dst_hbm, out_hbm, dacc, didx, ones, zb):
    cid = lax.axis_index("c")
    sid = lax.axis_index("s")
    w = sid * NC + cid
    for j in range(K // 16):
        ones[pl.ds(j * 16, 16)] = jnp.ones((16,), jnp.float32)
    for j in range(SLAB // 16):
        zb[pl.ds(j * 16, 16)] = jnp.zeros((16,), jnp.float32)
    pltpu.sync_copy(zb, dacc.at[pl.ds(sid * SLAB, SLAB)])
    pltpu.sync_copy(dst_hbm.at[w], didx)
    plsc.subcore_barrier()

    def body(c, _):
        pltpu.sync_copy(ones, dacc.at[didx.at[c]], add=True)
        return 0

    lax.fori_loop(0, CH, body, 0)
    plsc.subcore_barrier()
    pltpu.sync_copy(dacc.at[pl.ds(sid * SLAB, SLAB)],
                    out_hbm.at[pl.ds(cid * N2 + sid * SLAB, SLAB)])


@functools.partial(
    pl.kernel,
    out_type=jax.ShapeDtypeStruct((NC, NA, H), jnp.float32),
    mesh=_mesh,
    scratch_types=[
        pltpu.VMEM_SHARED((NA, H), jnp.float32),
        pltpu.VMEM((BCH, KA), jnp.int32),
        pltpu.VMEM((BCH, KA), jnp.int32),
        pltpu.VMEM((KA, H), jnp.float32),
        pltpu.VMEM((KA, H), jnp.float32),
        pltpu.VMEM((ZR, H), jnp.float32),
        pltpu.SemaphoreType.DMA,
        pltpu.SemaphoreType.DMA,
    ],
)
def _agg_call(y_hbm, src_hbm, dst_hbm, out_hbm,
              acc, sidx, didx, rows0, rows1, zbuf, sem0, sem1):
    cid = lax.axis_index("c")
    sid = lax.axis_index("s")
    w = sid * NC + cid
    for r in range(ZR):
        for j in range(H // 16):
            zbuf[r, pl.ds(j * 16, 16)] = jnp.zeros((16,), jnp.float32)

    def zslab(t, _):
        pltpu.sync_copy(zbuf, acc.at[pl.ds(sid * RPS + t * ZR, ZR)])
        return 0

    lax.fori_loop(0, RPS // ZR, zslab, 0)
    plsc.subcore_barrier()

    def batch(b, _):
        pltpu.sync_copy(src_hbm.at[w, b], sidx)
        pltpu.sync_copy(dst_hbm.at[w, b], didx)
        pltpu.async_copy(y_hbm.at[sidx.at[0]], rows0, sem0)
        for j in range(BCH):
            rs, sem = (rows0, sem0) if j % 2 == 0 else (rows1, sem1)
            rn, semn = (rows1, sem1) if j % 2 == 0 else (rows0, sem0)
            if j + 1 < BCH:
                pltpu.async_copy(y_hbm.at[sidx.at[j + 1]], rn, semn)
            pltpu.make_async_copy(y_hbm.at[sidx.at[j]], rs, sem).wait()
            pltpu.sync_copy(rs, acc.at[didx.at[j]], add=True)
        return 0

    lax.fori_loop(0, NB, batch, 0)
    plsc.subcore_barrier()
    pltpu.sync_copy(acc.at[pl.ds(sid * RPS, RPS)],
                    out_hbm.at[cid, pl.ds(sid * RPS, RPS)])


def _tc0_body(degs_ref, x_ref, w_ref, y_ref, dinv_ref):
    deg = degs_ref[0, :N] + degs_ref[1, :N] + 1.0
    dinv = lax.rsqrt(deg)
    dinv_ref[...] = dinv
    xw = jnp.dot(x_ref[...], w_ref[...], preferred_element_type=jnp.float32)
    y_ref[:N] = xw * dinv
    y_ref[N:] = jnp.zeros((NA - N, H), jnp.float32)


def _tc_first(degs, x, W0):
    return pl.pallas_call(
        _tc0_body,
        out_shape=[jax.ShapeDtypeStruct((NA, H), jnp.float32),
                   jax.ShapeDtypeStruct((N, 1), jnp.float32)],
    )(degs, x, W0)


def _tcmid_body(s_ref, y_ref, dinv_ref, b_ref, w_ref, ynext_ref):
    h = s_ref[0, :N] + s_ref[1, :N] + y_ref[:N]
    h = jnp.maximum(h * dinv_ref[...] + b_ref[...], 0.0)
    hw = jnp.dot(h, w_ref[...], preferred_element_type=jnp.float32)
    ynext_ref[:N] = hw * dinv_ref[...]
    ynext_ref[N:] = jnp.zeros((NA - N, H), jnp.float32)


def _tc_mid(s, y, dinv, b, Wn):
    return pl.pallas_call(
        _tcmid_body,
        out_shape=jax.ShapeDtypeStruct((NA, H), jnp.float32),
    )(s, y, dinv, b, Wn)


def _tcfin_body(s_ref, y_ref, dinv_ref, b_ref, batch_ref, wl_ref, bl_ref,
                out_ref):
    h = s_ref[0, :N] + s_ref[1, :N] + y_ref[:N]
    h = jnp.maximum(h * dinv_ref[...] + b_ref[...], 0.0)
    g = lax.broadcasted_iota(jnp.int32, (N, N_GRAPHS), 1)
    oh = (batch_ref[...] == g).astype(jnp.float32)
    sums = lax.dot_general(oh, h, (((0,), (0,)), ((), ())),
                           preferred_element_type=jnp.float32)
    counts = jnp.sum(oh, axis=0)
    pooled = sums / jnp.maximum(counts, 1.0)[:, None]
    out_ref[...] = jnp.dot(pooled, wl_ref[...],
                           preferred_element_type=jnp.float32) + bl_ref[...]


def _tc_final(s, y, dinv, b, batch, W_lin, b_lin):
    return pl.pallas_call(
        _tcfin_body,
        out_shape=jax.ShapeDtypeStruct((N_GRAPHS, N_PRED), jnp.float32),
    )(s, y, dinv, b, batch, W_lin, b_lin)


def _pad_edges(idx):
    arr = idx.reshape(NW, EPW)
    pad = jnp.full((NW, EPWP - EPW), N, dtype=idx.dtype)
    return jnp.concatenate([arr, pad], axis=1).reshape(NW, NB, BCH, KA)


def kernel(x, edge_index, batch, W0, b0, W1, b1, W2, b2, W_lin, b_lin):
    dst_deg = edge_index[1].reshape(NW, CH, K)
    src = _pad_edges(edge_index[0])
    dst = _pad_edges(edge_index[1])
    deg2 = _deg_call(dst_deg)
    degs = deg2.reshape(NC, N2)[:, :N].reshape(NC, N, 1)
    y0, dinv = _tc_first(degs, x, W0)
    s0 = _agg_call(y0, src, dst)
    y1 = _tc_mid(s0, y0, dinv, b0.reshape(1, H), W1)
    s1 = _agg_call(y1, src, dst)
    y2 = _tc_mid(s1, y1, dinv, b1.reshape(1, H), W2)
    s2 = _agg_call(y2, src, dst)
    return _tc_final(s2, y2, dinv, b2.reshape(1, H), batch.reshape(N, 1),
                     W_lin, b_lin.reshape(1, N_PRED))


# speedup vs baseline: 17.3347x; 1.8781x over previous
import functools

import jax
import jax.numpy as jnp
from jax import lax
from jax.experimental import pallas as pl
from jax.experimental.pallas import tpu as pltpu
from jax.experimental.pallas import tpu_sc as plsc

N = 10000
E = 320000
D = 128
H = 128
N_PRED = 8
N_GRAPHS = 64

NC = 2
NS = 16
NW = NC * NS
EPW = E // NW
K = 80
CH = EPW // K
N2 = 10240
SLAB = N2 // NS

NY = 10240
RPSY = NY // NS
QP = 4
NQ = NY // QP
RPQ = NQ // NS
KA = 128
CAP = 3072
CAPB = 3200
MAXCH = CAP // KA
ZR = 8
PADSRC = N

_mesh = plsc.VectorSubcoreMesh(core_axis_name="c", subcore_axis_name="s",
                               num_cores=NC, num_subcores=NS)


@functools.partial(
    pl.kernel,
    out_type=jax.ShapeDtypeStruct((NC * N2,), jnp.float32),
    mesh=_mesh,
    scratch_types=[
        pltpu.VMEM_SHARED((N2,), jnp.float32),
        pltpu.VMEM((CH, K), jnp.int32),
        pltpu.VMEM((K,), jnp.float32),
        pltpu.VMEM((SLAB,), jnp.float32),
    ],
)
def _deg_call(dst_hbm, out_hbm, dacc, didx, ones, zb):
    cid = lax.axis_index("c")
    sid = lax.axis_index("s")
    w = sid * NC + cid
    for j in range(K // 16):
        ones[pl.ds(j * 16, 16)] = jnp.ones((16,), jnp.float32)
    for j in range(SLAB // 16):
        zb[pl.ds(j * 16, 16)] = jnp.zeros((16,), jnp.float32)
    pltpu.sync_copy(zb, dacc.at[pl.ds(sid * SLAB, SLAB)])
    pltpu.sync_copy(dst_hbm.at[w], didx)
    plsc.subcore_barrier()

    def body(c, _):
        pltpu.sync_copy(ones, dacc.at[didx.at[c]], add=True)
        return 0

    lax.fori_loop(0, CH, body, 0)
    plsc.subcore_barrier()
    pltpu.sync_copy(dacc.at[pl.ds(sid * SLAB, SLAB)],
                    out_hbm.at[pl.ds(cid * N2 + sid * SLAB, SLAB)])


SCAP = 3200
WSZ = 4 * SCAP + 128
TRASH = 4 * SCAP
EPWP = 10112
NBLK = EPWP // 128


@functools.partial(
    pl.kernel,
    out_type=[jax.ShapeDtypeStruct((NW * QP * CAP,), jnp.int32),
              jax.ShapeDtypeStruct((NW * QP * CAP,), jnp.int32),
              jax.ShapeDtypeStruct((NW * 16,), jnp.int32)],
    mesh=_mesh,
    scratch_types=[
        pltpu.VMEM_SHARED((NS * WSZ,), jnp.int32),
        pltpu.VMEM_SHARED((NS * WSZ,), jnp.int32),
        pltpu.VMEM((EPWP,), jnp.int32),
        pltpu.VMEM((EPWP,), jnp.int32),
        pltpu.VMEM((128,), jnp.int32),
        pltpu.VMEM((128,), jnp.int32),
        pltpu.VMEM((128,), jnp.int32),
        pltpu.VMEM((CAP,), jnp.int32),
        pltpu.VMEM((CAP,), jnp.int32),
        pltpu.VMEM((16,), jnp.int32),
    ],
)
def _bucket_call(src_hbm, dst_hbm, bsrc_hbm, bdst_hbm, cnt_hbm,
                 bsp, bdp, svm, dvm, posb, svb, dvb, pfs, pfd, cbuf):
    cid = lax.axis_index("c")
    sid = lax.axis_index("s")
    w = sid * NC + cid
    wbase = sid * WSZ
    i16 = lax.iota(jnp.int32, 16)
    one = jnp.ones((16,), jnp.int32)

    def pfill(r, _):
        pfs[pl.ds(r * 16, 16)] = jnp.full((16,), PADSRC, jnp.int32)
        pfd[pl.ds(r * 16, 16)] = jnp.zeros((16,), jnp.int32)
        return 0

    lax.fori_loop(0, CAP // 16, pfill, 0)
    for q in range(QP):
        pltpu.sync_copy(pfs, bsp.at[pl.ds(wbase + q * SCAP, CAP)])
        pltpu.sync_copy(pfd, bdp.at[pl.ds(wbase + q * SCAP, CAP)])

    pltpu.sync_copy(src_hbm.at[pl.ds(w * EPW, EPW)], svm.at[pl.ds(0, EPW)])
    pltpu.sync_copy(dst_hbm.at[pl.ds(w * EPW, EPW)], dvm.at[pl.ds(0, EPW)])
    for t in range((EPWP - EPW) // 16):
        svm[pl.ds(EPW + t * 16, 16)] = jnp.full((16,), PADSRC, jnp.int32)
        dvm[pl.ds(EPW + t * 16, 16)] = jnp.full((16,), -1, jnp.int32)

    def block(b, carry):
        cnts = list(carry)
        for g in range(8):
            d = dvm[pl.ds(b * 128 + g * 16, 16)]
            sv = svm[pl.ds(b * 128 + g * 16, 16)]
            valid = d >= 0
            q = (jnp.where(d >= NQ, 1, 0) + jnp.where(d >= 2 * NQ, 1, 0)
                 + jnp.where(d >= 3 * NQ, 1, 0))
            v = jnp.where(valid, lax.shift_left(one, q * 8), 0)
            scan = v
            for k in [1, 2, 4, 8]:
                sk = jnp.take(scan, jnp.maximum(i16 - k, 0))
                scan = scan + jnp.where(i16 >= k, sk, 0)
            incl = lax.shift_right_logical(scan, q * 8) & 255
            csel = jnp.where(q == 0, cnts[0],
                             jnp.where(q == 1, cnts[1],
                                       jnp.where(q == 2, cnts[2], cnts[3])))
            pos = jnp.where(valid, q * SCAP + csel + incl - 1, TRASH + i16)
            posb[pl.ds(g * 16, 16)] = wbase + pos
            svb[pl.ds(g * 16, 16)] = sv
            dvb[pl.ds(g * 16, 16)] = d - q * NQ
            tot = jnp.take(scan, jnp.full((16,), 15, jnp.int32))
            for q2 in range(QP):
                cnts[q2] = cnts[q2] + (
                    lax.shift_right_logical(tot, q2 * 8) & 255)
        pltpu.sync_copy(svb, bsp.at[posb])
        pltpu.sync_copy(dvb, bdp.at[posb])
        return tuple(cnts)

    z = jnp.zeros((16,), jnp.int32)
    cnts = lax.fori_loop(0, NBLK, block, (z, z, z, z))

    vec = jnp.zeros((16,), jnp.int32)
    for q in range(QP):
        ch = lax.shift_right_logical(cnts[q] + (KA - 1), 7)
        ch = jnp.minimum(ch, MAXCH)
        vec = vec + jnp.where(i16 == q, ch, 0)
    cbuf[pl.ds(0, 16)] = vec
    pltpu.sync_copy(cbuf, cnt_hbm.at[pl.ds(w * 16, 16)])
    for q in range(QP):
        base = (w * QP + q) * CAP
        pltpu.sync_copy(bsp.at[pl.ds(wbase + q * SCAP, CAP)],
                        bsrc_hbm.at[pl.ds(base, CAP)])
        pltpu.sync_copy(bdp.at[pl.ds(wbase + q * SCAP, CAP)],
                        bdst_hbm.at[pl.ds(base, CAP)])


@functools.partial(
    pl.kernel,
    out_type=jax.ShapeDtypeStruct((NC, NY, H), jnp.float32),
    mesh=_mesh,
    scratch_types=[
        pltpu.VMEM_SHARED((NY, H), jnp.float32),
        pltpu.VMEM_SHARED((NQ, H), jnp.float32),
        pltpu.VMEM((MAXCH, KA), jnp.int32),
        pltpu.VMEM((MAXCH, KA), jnp.int32),
        pltpu.VMEM((KA, H), jnp.float32),
        pltpu.VMEM((ZR, H), jnp.float32),
        pltpu.VMEM((16,), jnp.int32),
        pltpu.SemaphoreType.DMA,
    ],
)
def _agg_call(y_hbm, bsrc_hbm, bdst_hbm, cnt_hbm, out_hbm,
              ysp, acc, sidx, didx, rows, zbuf, cvm, sem):
    cid = lax.axis_index("c")
    sid = lax.axis_index("s")
    w = sid * NC + cid
    for r in range(ZR):
        for j in range(H // 16):
            zbuf[r, pl.ds(j * 16, 16)] = jnp.zeros((16,), jnp.float32)
    pltpu.sync_copy(y_hbm.at[pl.ds(sid * RPSY, RPSY)],
                    ysp.at[pl.ds(sid * RPSY, RPSY)])
    pltpu.sync_copy(cnt_hbm.at[pl.ds(w * 16, 16)], cvm)
    nchs = cvm[pl.ds(0, 16)]

    for q in range(QP):
        def zslab(t, _):
            pltpu.sync_copy(zbuf, acc.at[pl.ds(sid * RPQ + t * ZR, ZR)])
            return 0

        lax.fori_loop(0, RPQ // ZR, zslab, 0)
        pltpu.sync_copy(bsrc_hbm.at[w, q], sidx)
        pltpu.sync_copy(bdst_hbm.at[w, q], didx)
        plsc.subcore_barrier()
        nch = nchs[q]

        def chunk(c, _):
            pltpu.async_copy(ysp.at[sidx.at[c]], rows, sem).wait()
            pltpu.sync_copy(rows, acc.at[didx.at[c]], add=True)
            return 0

        lax.fori_loop(0, nch, chunk, 0)
        plsc.subcore_barrier()
        pltpu.sync_copy(acc.at[pl.ds(sid * RPQ, RPQ)],
                        out_hbm.at[cid, pl.ds(q * NQ + sid * RPQ, RPQ)])
        plsc.subcore_barrier()


def _tc0_body(degs_ref, x_ref, w_ref, y_ref, dinv_ref):
    deg = degs_ref[0, :N] + degs_ref[1, :N] + 1.0
    dinv = lax.rsqrt(deg)
    dinv_ref[...] = dinv
    xw = jnp.dot(x_ref[...], w_ref[...], preferred_element_type=jnp.float32)
    y_ref[:N] = xw * dinv
    y_ref[N:] = jnp.zeros((NY - N, H), jnp.float32)


def _tc_first(degs, x, W0):
    return pl.pallas_call(
        _tc0_body,
        out_shape=[jax.ShapeDtypeStruct((NY, H), jnp.float32),
                   jax.ShapeDtypeStruct((N, 1), jnp.float32)],
    )(degs, x, W0)


def _tcmid_body(s_ref, y_ref, dinv_ref, b_ref, w_ref, ynext_ref):
    h = s_ref[0, :N] + s_ref[1, :N] + y_ref[:N]
    h = jnp.maximum(h * dinv_ref[...] + b_ref[...], 0.0)
    hw = jnp.dot(h, w_ref[...], preferred_element_type=jnp.float32)
    ynext_ref[:N] = hw * dinv_ref[...]
    ynext_ref[N:] = jnp.zeros((NY - N, H), jnp.float32)


def _tc_mid(s, y, dinv, b, Wn):
    return pl.pallas_call(
        _tcmid_body,
        out_shape=jax.ShapeDtypeStruct((NY, H), jnp.float32),
    )(s, y, dinv, b, Wn)


def _tcfin_body(s_ref, y_ref, dinv_ref, b_ref, batch_ref, wl_ref, bl_ref,
                out_ref):
    h = s_ref[0, :N] + s_ref[1, :N] + y_ref[:N]
    h = jnp.maximum(h * dinv_ref[...] + b_ref[...], 0.0)
    g = lax.broadcasted_iota(jnp.int32, (N, N_GRAPHS), 1)
    oh = (batch_ref[...] == g).astype(jnp.float32)
    sums = lax.dot_general(oh, h, (((0,), (0,)), ((), ())),
                           preferred_element_type=jnp.float32)
    counts = jnp.sum(oh, axis=0)
    pooled = sums / jnp.maximum(counts, 1.0)[:, None]
    out_ref[...] = jnp.dot(pooled, wl_ref[...],
                           preferred_element_type=jnp.float32) + bl_ref[...]


def _tc_final(s, y, dinv, b, batch, W_lin, b_lin):
    return pl.pallas_call(
        _tcfin_body,
        out_shape=jax.ShapeDtypeStruct((N_GRAPHS, N_PRED), jnp.float32),
    )(s, y, dinv, b, batch, W_lin, b_lin)


def kernel(x, edge_index, batch, W0, b0, W1, b1, W2, b2, W_lin, b_lin):
    srcf = edge_index[0]
    dstf = edge_index[1]
    dst_deg = dstf.reshape(NW, CH, K)
    bsrc, bdst, cnts = _bucket_call(srcf, dstf)
    bsrc4 = bsrc.reshape(NW, QP, MAXCH, KA)
    bdst4 = bdst.reshape(NW, QP, MAXCH, KA)
    deg2 = _deg_call(dst_deg)
    degs = deg2.reshape(NC, N2)[:, :N].reshape(NC, N, 1)
    y0, dinv = _tc_first(degs, x, W0)
    s0 = _agg_call(y0, bsrc4, bdst4, cnts)
    y1 = _tc_mid(s0, y0, dinv, b0.reshape(1, H), W1)
    s1 = _agg_call(y1, bsrc4, bdst4, cnts)
    y2 = _tc_mid(s1, y1, dinv, b1.reshape(1, H), W2)
    s2 = _agg_call(y2, bsrc4, bdst4, cnts)
    return _tc_final(s2, y2, dinv, b2.reshape(1, H), batch.reshape(N, 1),
                     W_lin, b_lin.reshape(1, N_PRED))


# speedup vs baseline: 19.4051x; 1.1194x over previous
import functools

import jax
import jax.numpy as jnp
from jax import lax
from jax.experimental import pallas as pl
from jax.experimental.pallas import tpu as pltpu
from jax.experimental.pallas import tpu_sc as plsc

N = 10000
E = 320000
D = 128
H = 128
N_PRED = 8
N_GRAPHS = 64

NC = 2
NS = 16
NW = NC * NS
EPW = E // NW
K = 80
CH = EPW // K
N2 = 10240
SLAB = N2 // NS

NY = 10240
YR = 10112
RPSY = YR // NS
QP = 4
NQ = NY // QP
RPQ = NQ // NS
KA = 128
CAP = 3072
CAPB = 3200
MAXCH = CAP // KA
KA2 = 64
MAXCH2 = CAP // KA2
ZR = 4
PADSRC = N

_mesh = plsc.VectorSubcoreMesh(core_axis_name="c", subcore_axis_name="s",
                               num_cores=NC, num_subcores=NS)


@functools.partial(
    pl.kernel,
    out_type=jax.ShapeDtypeStruct((NC * N2,), jnp.float32),
    mesh=_mesh,
    scratch_types=[
        pltpu.VMEM_SHARED((N2,), jnp.float32),
        pltpu.VMEM((CH, K), jnp.int32),
        pltpu.VMEM((K,), jnp.float32),
        pltpu.VMEM((SLAB,), jnp.float32),
    ],
)
def _deg_call(dst_hbm, out_hbm, dacc, didx, ones, zb):
    cid = lax.axis_index("c")
    sid = lax.axis_index("s")
    w = sid * NC + cid
    for j in range(K // 16):
        ones[pl.ds(j * 16, 16)] = jnp.ones((16,), jnp.float32)
    for j in range(SLAB // 16):
        zb[pl.ds(j * 16, 16)] = jnp.zeros((16,), jnp.float32)
    pltpu.sync_copy(zb, dacc.at[pl.ds(sid * SLAB, SLAB)])
    pltpu.sync_copy(dst_hbm.at[w], didx)
    plsc.subcore_barrier()

    def body(c, _):
        pltpu.sync_copy(ones, dacc.at[didx.at[c]], add=True)
        return 0

    lax.fori_loop(0, CH, body, 0)
    plsc.subcore_barrier()
    pltpu.sync_copy(dacc.at[pl.ds(sid * SLAB, SLAB)],
                    out_hbm.at[pl.ds(cid * N2 + sid * SLAB, SLAB)])


SCAP = 3200
WSZ = 4 * SCAP + 128
TRASH = 4 * SCAP
EPWP = 10112
NBLK = EPWP // 128


@functools.partial(
    pl.kernel,
    out_type=[jax.ShapeDtypeStruct((NW * QP * CAP,), jnp.int32),
              jax.ShapeDtypeStruct((NW * QP * CAP,), jnp.int32),
              jax.ShapeDtypeStruct((NW * 16,), jnp.int32)],
    mesh=_mesh,
    scratch_types=[
        pltpu.VMEM_SHARED((NS * WSZ,), jnp.int32),
        pltpu.VMEM_SHARED((NS * WSZ,), jnp.int32),
        pltpu.VMEM((EPWP,), jnp.int32),
        pltpu.VMEM((EPWP,), jnp.int32),
        pltpu.VMEM((128,), jnp.int32),
        pltpu.VMEM((128,), jnp.int32),
        pltpu.VMEM((128,), jnp.int32),
        pltpu.VMEM((CAP,), jnp.int32),
        pltpu.VMEM((CAP,), jnp.int32),
        pltpu.VMEM((16,), jnp.int32),
    ],
)
def _bucket_call(src_hbm, dst_hbm, bsrc_hbm, bdst_hbm, cnt_hbm,
                 bsp, bdp, svm, dvm, posb, svb, dvb, pfs, pfd, cbuf):
    cid = lax.axis_index("c")
    sid = lax.axis_index("s")
    w = sid * NC + cid
    wbase = sid * WSZ
    i16 = lax.iota(jnp.int32, 16)
    one = jnp.ones((16,), jnp.int32)

    def pfill(r, _):
        pfs[pl.ds(r * 16, 16)] = jnp.full((16,), PADSRC, jnp.int32)
        pfd[pl.ds(r * 16, 16)] = jnp.zeros((16,), jnp.int32)
        return 0

    lax.fori_loop(0, CAP // 16, pfill, 0)
    for q in range(QP):
        pltpu.sync_copy(pfs, bsp.at[pl.ds(wbase + q * SCAP, CAP)])
        pltpu.sync_copy(pfd, bdp.at[pl.ds(wbase + q * SCAP, CAP)])

    pltpu.sync_copy(src_hbm.at[pl.ds(w * EPW, EPW)], svm.at[pl.ds(0, EPW)])
    pltpu.sync_copy(dst_hbm.at[pl.ds(w * EPW, EPW)], dvm.at[pl.ds(0, EPW)])
    for t in range((EPWP - EPW) // 16):
        svm[pl.ds(EPW + t * 16, 16)] = jnp.full((16,), PADSRC, jnp.int32)
        dvm[pl.ds(EPW + t * 16, 16)] = jnp.full((16,), -1, jnp.int32)

    def block(b, carry):
        cnts = list(carry)
        for g in range(8):
            d = dvm[pl.ds(b * 128 + g * 16, 16)]
            sv = svm[pl.ds(b * 128 + g * 16, 16)]
            valid = d >= 0
            q = (jnp.where(d >= NQ, 1, 0) + jnp.where(d >= 2 * NQ, 1, 0)
                 + jnp.where(d >= 3 * NQ, 1, 0))
            v = jnp.where(valid, lax.shift_left(one, q * 8), 0)
            scan = v
            for k in [1, 2, 4, 8]:
                sk = jnp.take(scan, jnp.maximum(i16 - k, 0))
                scan = scan + jnp.where(i16 >= k, sk, 0)
            incl = lax.shift_right_logical(scan, q * 8) & 255
            csel = jnp.where(q == 0, cnts[0],
                             jnp.where(q == 1, cnts[1],
                                       jnp.where(q == 2, cnts[2], cnts[3])))
            pos = jnp.where(valid, q * SCAP + csel + incl - 1, TRASH + i16)
            posb[pl.ds(g * 16, 16)] = wbase + pos
            svb[pl.ds(g * 16, 16)] = sv
            dvb[pl.ds(g * 16, 16)] = d - q * NQ
            tot = jnp.take(scan, jnp.full((16,), 15, jnp.int32))
            for q2 in range(QP):
                cnts[q2] = cnts[q2] + (
                    lax.shift_right_logical(tot, q2 * 8) & 255)
        pltpu.sync_copy(svb, bsp.at[posb])
        pltpu.sync_copy(dvb, bdp.at[posb])
        return tuple(cnts)

    z = jnp.zeros((16,), jnp.int32)
    cnts = lax.fori_loop(0, NBLK, block, (z, z, z, z))

    vec = jnp.zeros((16,), jnp.int32)
    for q in range(QP):
        ch = lax.shift_right_logical(cnts[q] + (KA2 - 1), 6)
        ch = jnp.minimum(ch, MAXCH2)
        vec = vec + jnp.where(i16 == q, ch, 0)
    cbuf[pl.ds(0, 16)] = vec
    pltpu.sync_copy(cbuf, cnt_hbm.at[pl.ds(w * 16, 16)])
    for q in range(QP):
        base = (w * QP + q) * CAP
        pltpu.sync_copy(bsp.at[pl.ds(wbase + q * SCAP, CAP)],
                        bsrc_hbm.at[pl.ds(base, CAP)])
        pltpu.sync_copy(bdp.at[pl.ds(wbase + q * SCAP, CAP)],
                        bdst_hbm.at[pl.ds(base, CAP)])


@functools.partial(
    pl.kernel,
    out_type=jax.ShapeDtypeStruct((NC, NY, H), jnp.float32),
    mesh=_mesh,
    scratch_types=[
        pltpu.VMEM_SHARED((YR, H), jnp.float32),
        pltpu.VMEM_SHARED((NQ, H), jnp.float32),
        pltpu.VMEM((MAXCH2, KA2), jnp.int32),
        pltpu.VMEM((MAXCH2, KA2), jnp.int32),
        pltpu.VMEM((KA2, H), jnp.float32),
        pltpu.VMEM((KA2, H), jnp.float32),
        pltpu.VMEM((ZR, H), jnp.float32),
        pltpu.VMEM((16,), jnp.int32),
        pltpu.SemaphoreType.DMA,
        pltpu.SemaphoreType.DMA,
    ],
)
def _agg_call(y_hbm, bsrc_hbm, bdst_hbm, cnt_hbm, out_hbm,
              ysp, acc, sidx, didx, rows0, rows1, zbuf, cvm, sem0, sem1):
    cid = lax.axis_index("c")
    sid = lax.axis_index("s")
    w = sid * NC + cid
    for r in range(ZR):
        for j in range(H // 16):
            zbuf[r, pl.ds(j * 16, 16)] = jnp.zeros((16,), jnp.float32)
    pltpu.sync_copy(y_hbm.at[pl.ds(sid * RPSY, RPSY)],
                    ysp.at[pl.ds(sid * RPSY, RPSY)])
    pltpu.sync_copy(cnt_hbm.at[pl.ds(w * 16, 16)], cvm)
    nchs = cvm[pl.ds(0, 16)]

    for q in range(QP):
        def zslab(t, _):
            pltpu.sync_copy(zbuf, acc.at[pl.ds(sid * RPQ + t * ZR, ZR)])
            return 0

        lax.fori_loop(0, RPQ // ZR, zslab, 0)
        pltpu.sync_copy(bsrc_hbm.at[w, q], sidx)
        pltpu.sync_copy(bdst_hbm.at[w, q], didx)
        plsc.subcore_barrier()
        nch = nchs[q]

        @pl.when(nch > 0)
        def _():
            pltpu.async_copy(ysp.at[sidx.at[0]], rows0, sem0)

        def chunk(c, _):
            nxt = c + 1
            even_nxt = (nxt & 1) == 0

            @pl.when((nxt < nch) & even_nxt)
            def _():
                pltpu.async_copy(ysp.at[sidx.at[nxt]], rows0, sem0)

            @pl.when((nxt < nch) & jnp.logical_not(even_nxt))
            def _():
                pltpu.async_copy(ysp.at[sidx.at[nxt]], rows1, sem1)

            @pl.when((c & 1) == 0)
            def _():
                pltpu.make_async_copy(ysp.at[sidx.at[c]], rows0, sem0).wait()
                pltpu.sync_copy(rows0, acc.at[didx.at[c]], add=True)

            @pl.when((c & 1) == 1)
            def _():
                pltpu.make_async_copy(ysp.at[sidx.at[c]], rows1, sem1).wait()
                pltpu.sync_copy(rows1, acc.at[didx.at[c]], add=True)

            return 0

        lax.fori_loop(0, nch, chunk, 0)
        plsc.subcore_barrier()
        pltpu.sync_copy(acc.at[pl.ds(sid * RPQ, RPQ)],
                        out_hbm.at[cid, pl.ds(q * NQ + sid * RPQ, RPQ)])
        plsc.subcore_barrier()


def _tc0_body(degs_ref, x_ref, w_ref, y_ref, dinv_ref):
    deg = degs_ref[0, :N] + degs_ref[1, :N] + 1.0
    dinv = lax.rsqrt(deg)
    dinv_ref[...] = dinv
    xw = jnp.dot(x_ref[...], w_ref[...], preferred_element_type=jnp.float32)
    y_ref[:N] = xw * dinv
    y_ref[N:] = jnp.zeros((YR - N, H), jnp.float32)


def _tc_first(degs, x, W0):
    return pl.pallas_call(
        _tc0_body,
        out_shape=[jax.ShapeDtypeStruct((YR, H), jnp.float32),
                   jax.ShapeDtypeStruct((N, 1), jnp.float32)],
    )(degs, x, W0)


def _tcmid_body(s_ref, y_ref, dinv_ref, b_ref, w_ref, ynext_ref):
    h = s_ref[0, :N] + s_ref[1, :N] + y_ref[:N]
    h = jnp.maximum(h * dinv_ref[...] + b_ref[...], 0.0)
    hw = jnp.dot(h, w_ref[...], preferred_element_type=jnp.float32)
    ynext_ref[:N] = hw * dinv_ref[...]
    ynext_ref[N:] = jnp.zeros((YR - N, H), jnp.float32)


def _tc_mid(s, y, dinv, b, Wn):
    return pl.pallas_call(
        _tcmid_body,
        out_shape=jax.ShapeDtypeStruct((YR, H), jnp.float32),
    )(s, y, dinv, b, Wn)


def _tcfin_body(s_ref, y_ref, dinv_ref, b_ref, batch_ref, wl_ref, bl_ref,
                out_ref):
    h = s_ref[0, :N] + s_ref[1, :N] + y_ref[:N]
    h = jnp.maximum(h * dinv_ref[...] + b_ref[...], 0.0)
    g = lax.broadcasted_iota(jnp.int32, (N, N_GRAPHS), 1)
    oh = (batch_ref[...] == g).astype(jnp.float32)
    sums = lax.dot_general(oh, h, (((0,), (0,)), ((), ())),
                           preferred_element_type=jnp.float32)
    counts = jnp.sum(oh, axis=0)
    pooled = sums / jnp.maximum(counts, 1.0)[:, None]
    out_ref[...] = jnp.dot(pooled, wl_ref[...],
                           preferred_element_type=jnp.float32) + bl_ref[...]


def _tc_final(s, y, dinv, b, batch, W_lin, b_lin):
    return pl.pallas_call(
        _tcfin_body,
        out_shape=jax.ShapeDtypeStruct((N_GRAPHS, N_PRED), jnp.float32),
    )(s, y, dinv, b, batch, W_lin, b_lin)


def kernel(x, edge_index, batch, W0, b0, W1, b1, W2, b2, W_lin, b_lin):
    srcf = edge_index[0]
    dstf = edge_index[1]
    dst_deg = dstf.reshape(NW, CH, K)
    bsrc, bdst, cnts = _bucket_call(srcf, dstf)
    bsrc4 = bsrc.reshape(NW, QP, MAXCH2, KA2)
    bdst4 = bdst.reshape(NW, QP, MAXCH2, KA2)
    deg2 = _deg_call(dst_deg)
    degs = deg2.reshape(NC, N2)[:, :N].reshape(NC, N, 1)
    y0, dinv = _tc_first(degs, x, W0)
    s0 = _agg_call(y0, bsrc4, bdst4, cnts)
    y1 = _tc_mid(s0, y0, dinv, b0.reshape(1, H), W1)
    s1 = _agg_call(y1, bsrc4, bdst4, cnts)
    y2 = _tc_mid(s1, y1, dinv, b1.reshape(1, H), W2)
    s2 = _agg_call(y2, bsrc4, bdst4, cnts)
    return _tc_final(s2, y2, dinv, b2.reshape(1, H), batch.reshape(N, 1),
                     W_lin, b_lin.reshape(1, N_PRED))


# speedup vs baseline: 19.8947x; 1.0252x over previous
import functools

import jax
import jax.numpy as jnp
from jax import lax
from jax.experimental import pallas as pl
from jax.experimental.pallas import tpu as pltpu
from jax.experimental.pallas import tpu_sc as plsc

N = 10000
E = 320000
D = 128
H = 128
N_PRED = 8
N_GRAPHS = 64

NC = 2
NS = 16
NW = NC * NS
EPW = E // NW
K = 80
CH = EPW // K
N2 = 10240
SLAB = N2 // NS

NY = 10240
YR = 10112
RPSY = YR // NS
QP = 4
NQ = NY // QP
RPQ = NQ // NS
KA = 128
CAP = 3072
CAPB = 3200
MAXCH = CAP // KA
KA2 = 64
MAXCH2 = CAP // KA2
ZR = 4
PADSRC = N

_mesh = plsc.VectorSubcoreMesh(core_axis_name="c", subcore_axis_name="s",
                               num_cores=NC, num_subcores=NS)


@functools.partial(
    pl.kernel,
    out_type=jax.ShapeDtypeStruct((NC * N2,), jnp.float32),
    mesh=_mesh,
    scratch_types=[
        pltpu.VMEM_SHARED((N2,), jnp.float32),
        pltpu.VMEM((CH, K), jnp.int32),
        pltpu.VMEM((K,), jnp.float32),
        pltpu.VMEM((SLAB,), jnp.float32),
    ],
)
def _deg_call(dst_hbm, out_hbm, dacc, didx, ones, zb):
    cid = lax.axis_index("c")
    sid = lax.axis_index("s")
    w = sid * NC + cid
    for j in range(K // 16):
        ones[pl.ds(j * 16, 16)] = jnp.ones((16,), jnp.float32)
    for j in range(SLAB // 16):
        zb[pl.ds(j * 16, 16)] = jnp.zeros((16,), jnp.float32)
    pltpu.sync_copy(zb, dacc.at[pl.ds(sid * SLAB, SLAB)])
    pltpu.sync_copy(dst_hbm.at[w], didx)
    plsc.subcore_barrier()

    def body(c, _):
        pltpu.sync_copy(ones, dacc.at[didx.at[c]], add=True)
        return 0

    lax.fori_loop(0, CH, body, 0)
    plsc.subcore_barrier()
    pltpu.sync_copy(dacc.at[pl.ds(sid * SLAB, SLAB)],
                    out_hbm.at[pl.ds(cid * N2 + sid * SLAB, SLAB)])


SCAP = 3200
WSZ = 4 * SCAP + 128
TRASH = 4 * SCAP
EPWP = 10112
NBLK = EPWP // 128


@functools.partial(
    pl.kernel,
    out_type=[jax.ShapeDtypeStruct((NW * QP * CAP,), jnp.int32),
              jax.ShapeDtypeStruct((NW * QP * CAP,), jnp.int32),
              jax.ShapeDtypeStruct((NW * 16,), jnp.int32)],
    mesh=_mesh,
    scratch_types=[
        pltpu.VMEM_SHARED((NS * WSZ,), jnp.int32),
        pltpu.VMEM_SHARED((NS * WSZ,), jnp.int32),
        pltpu.VMEM((EPWP,), jnp.int32),
        pltpu.VMEM((EPWP,), jnp.int32),
        pltpu.VMEM((128,), jnp.int32),
        pltpu.VMEM((128,), jnp.int32),
        pltpu.VMEM((128,), jnp.int32),
        pltpu.VMEM((CAP,), jnp.int32),
        pltpu.VMEM((CAP,), jnp.int32),
        pltpu.VMEM((16,), jnp.int32),
    ],
)
def _bucket_call(src_hbm, dst_hbm, bsrc_hbm, bdst_hbm, cnt_hbm,
                 bsp, bdp, svm, dvm, posb, svb, dvb, pfs, pfd, cbuf):
    cid = lax.axis_index("c")
    sid = lax.axis_index("s")
    w = sid * NC + cid
    wbase = sid * WSZ
    i16 = lax.iota(jnp.int32, 16)
    one = jnp.ones((16,), jnp.int32)

    def pfill(r, _):
        pfs[pl.ds(r * 16, 16)] = jnp.full((16,), PADSRC, jnp.int32)
        pfd[pl.ds(r * 16, 16)] = jnp.zeros((16,), jnp.int32)
        return 0

    lax.fori_loop(0, CAP // 16, pfill, 0)
    for q in range(QP):
        pltpu.sync_copy(pfs, bsp.at[pl.ds(wbase + q * SCAP, CAP)])
        pltpu.sync_copy(pfd, bdp.at[pl.ds(wbase + q * SCAP, CAP)])

    pltpu.sync_copy(src_hbm.at[pl.ds(w * EPW, EPW)], svm.at[pl.ds(0, EPW)])
    pltpu.sync_copy(dst_hbm.at[pl.ds(w * EPW, EPW)], dvm.at[pl.ds(0, EPW)])
    for t in range((EPWP - EPW) // 16):
        svm[pl.ds(EPW + t * 16, 16)] = jnp.full((16,), PADSRC, jnp.int32)
        dvm[pl.ds(EPW + t * 16, 16)] = jnp.full((16,), -1, jnp.int32)

    def block(b, carry):
        cnts = list(carry)
        for g in range(8):
            d = dvm[pl.ds(b * 128 + g * 16, 16)]
            sv = svm[pl.ds(b * 128 + g * 16, 16)]
            valid = d >= 0
            q = (jnp.where(d >= NQ, 1, 0) + jnp.where(d >= 2 * NQ, 1, 0)
                 + jnp.where(d >= 3 * NQ, 1, 0))
            v = jnp.where(valid, lax.shift_left(one, q * 8), 0)
            scan = v
            for k in [1, 2, 4, 8]:
                sk = jnp.take(scan, jnp.maximum(i16 - k, 0))
                scan = scan + jnp.where(i16 >= k, sk, 0)
            incl = lax.shift_right_logical(scan, q * 8) & 255
            csel = jnp.where(q == 0, cnts[0],
                             jnp.where(q == 1, cnts[1],
                                       jnp.where(q == 2, cnts[2], cnts[3])))
            pos = jnp.where(valid, q * SCAP + csel + incl - 1, TRASH + i16)
            posb[pl.ds(g * 16, 16)] = wbase + pos
            svb[pl.ds(g * 16, 16)] = sv
            dvb[pl.ds(g * 16, 16)] = d - q * NQ
            tot = jnp.take(scan, jnp.full((16,), 15, jnp.int32))
            for q2 in range(QP):
                cnts[q2] = cnts[q2] + (
                    lax.shift_right_logical(tot, q2 * 8) & 255)
        pltpu.sync_copy(svb, bsp.at[posb])
        pltpu.sync_copy(dvb, bdp.at[posb])
        return tuple(cnts)

    z = jnp.zeros((16,), jnp.int32)
    cnts = lax.fori_loop(0, NBLK, block, (z, z, z, z))

    vec = jnp.zeros((16,), jnp.int32)
    for q in range(QP):
        ch = lax.shift_right_logical(cnts[q] + (KA2 - 1), 6)
        ch = jnp.minimum(ch, MAXCH2)
        vec = vec + jnp.where(i16 == q, ch, 0)
    cbuf[pl.ds(0, 16)] = vec
    pltpu.sync_copy(cbuf, cnt_hbm.at[pl.ds(w * 16, 16)])
    for q in range(QP):
        base = (w * QP + q) * CAP
        pltpu.sync_copy(bsp.at[pl.ds(wbase + q * SCAP, CAP)],
                        bsrc_hbm.at[pl.ds(base, CAP)])
        pltpu.sync_copy(bdp.at[pl.ds(wbase + q * SCAP, CAP)],
                        bdst_hbm.at[pl.ds(base, CAP)])


@functools.partial(
    pl.kernel,
    out_type=jax.ShapeDtypeStruct((NC, NY, H), jnp.float32),
    mesh=_mesh,
    scratch_types=[
        pltpu.VMEM_SHARED((YR, H), jnp.float32),
        pltpu.VMEM_SHARED((NQ, H), jnp.float32),
        pltpu.VMEM((MAXCH2, KA2), jnp.int32),
        pltpu.VMEM((MAXCH2, KA2), jnp.int32),
        pltpu.VMEM((KA2, H), jnp.float32),
        pltpu.VMEM((KA2, H), jnp.float32),
        pltpu.VMEM((16,), jnp.int32),
        pltpu.SemaphoreType.DMA,
        pltpu.SemaphoreType.DMA,
    ],
)
def _agg_call(y_hbm, bsrc_hbm, bdst_hbm, cnt_hbm, out_hbm,
              ysp, acc, sidx, didx, rows0, rows1, cvm, sem0, sem1):
    cid = lax.axis_index("c")
    sid = lax.axis_index("s")
    w = sid * NC + cid
    pltpu.sync_copy(y_hbm.at[pl.ds(sid * RPSY, RPSY)],
                    ysp.at[pl.ds(sid * RPSY, RPSY)])
    pltpu.sync_copy(cnt_hbm.at[pl.ds(w * 16, 16)], cvm)
    nchs = cvm[pl.ds(0, 16)]

    for q in range(QP):
        def zrows(r, _):
            for j in range(H // 16):
                rows0[r, pl.ds(j * 16, 16)] = jnp.zeros((16,), jnp.float32)
                rows1[r, pl.ds(j * 16, 16)] = jnp.zeros((16,), jnp.float32)
            return 0

        lax.fori_loop(0, KA2, zrows, 0)
        base = sid * RPQ
        pltpu.sync_copy(rows0, acc.at[pl.ds(base, KA2)])
        pltpu.sync_copy(rows1, acc.at[pl.ds(base + KA2, KA2)])
        pltpu.sync_copy(rows0.at[pl.ds(0, RPQ - 2 * KA2)],
                        acc.at[pl.ds(base + 2 * KA2, RPQ - 2 * KA2)])
        pltpu.sync_copy(bsrc_hbm.at[w, q], sidx)
        pltpu.sync_copy(bdst_hbm.at[w, q], didx)
        plsc.subcore_barrier()
        nch = nchs[q]

        @pl.when(nch > 0)
        def _():
            pltpu.async_copy(ysp.at[sidx.at[0]], rows0, sem0)

        def chunk(c, _):
            nxt = c + 1
            even_nxt = (nxt & 1) == 0

            @pl.when((nxt < nch) & even_nxt)
            def _():
                pltpu.async_copy(ysp.at[sidx.at[nxt]], rows0, sem0)

            @pl.when((nxt < nch) & jnp.logical_not(even_nxt))
            def _():
                pltpu.async_copy(ysp.at[sidx.at[nxt]], rows1, sem1)

            @pl.when((c & 1) == 0)
            def _():
                pltpu.make_async_copy(ysp.at[sidx.at[c]], rows0, sem0).wait()
                pltpu.sync_copy(rows0, acc.at[didx.at[c]], add=True)

            @pl.when((c & 1) == 1)
            def _():
                pltpu.make_async_copy(ysp.at[sidx.at[c]], rows1, sem1).wait()
                pltpu.sync_copy(rows1, acc.at[didx.at[c]], add=True)

            return 0

        lax.fori_loop(0, nch, chunk, 0)
        plsc.subcore_barrier()
        pltpu.sync_copy(acc.at[pl.ds(sid * RPQ, RPQ)],
                        out_hbm.at[cid, pl.ds(q * NQ + sid * RPQ, RPQ)])
        plsc.subcore_barrier()


def _tc0_body(degs_ref, x_ref, w_ref, y_ref, dinv_ref):
    deg = degs_ref[0, :N] + degs_ref[1, :N] + 1.0
    dinv = lax.rsqrt(deg)
    dinv_ref[...] = dinv
    xw = jnp.dot(x_ref[...], w_ref[...], preferred_element_type=jnp.float32)
    y_ref[:N] = xw * dinv
    y_ref[N:] = jnp.zeros((YR - N, H), jnp.float32)


def _tc_first(degs, x, W0):
    return pl.pallas_call(
        _tc0_body,
        out_shape=[jax.ShapeDtypeStruct((YR, H), jnp.float32),
                   jax.ShapeDtypeStruct((N, 1), jnp.float32)],
    )(degs, x, W0)


def _tcmid_body(s_ref, y_ref, dinv_ref, b_ref, w_ref, ynext_ref):
    h = s_ref[0, :N] + s_ref[1, :N] + y_ref[:N]
    h = jnp.maximum(h * dinv_ref[...] + b_ref[...], 0.0)
    hw = jnp.dot(h, w_ref[...], preferred_element_type=jnp.float32)
    ynext_ref[:N] = hw * dinv_ref[...]
    ynext_ref[N:] = jnp.zeros((YR - N, H), jnp.float32)


def _tc_mid(s, y, dinv, b, Wn):
    return pl.pallas_call(
        _tcmid_body,
        out_shape=jax.ShapeDtypeStruct((YR, H), jnp.float32),
    )(s, y, dinv, b, Wn)


def _tcfin_body(s_ref, y_ref, dinv_ref, b_ref, batch_ref, wl_ref, bl_ref,
                out_ref):
    h = s_ref[0, :N] + s_ref[1, :N] + y_ref[:N]
    h = jnp.maximum(h * dinv_ref[...] + b_ref[...], 0.0)
    g = lax.broadcasted_iota(jnp.int32, (N, N_GRAPHS), 1)
    oh = (batch_ref[...] == g).astype(jnp.float32)
    sums = lax.dot_general(oh, h, (((0,), (0,)), ((), ())),
                           preferred_element_type=jnp.float32)
    counts = jnp.sum(oh, axis=0)
    pooled = sums / jnp.maximum(counts, 1.0)[:, None]
    out_ref[...] = jnp.dot(pooled, wl_ref[...],
                           preferred_element_type=jnp.float32) + bl_ref[...]


def _tc_final(s, y, dinv, b, batch, W_lin, b_lin):
    return pl.pallas_call(
        _tcfin_body,
        out_shape=jax.ShapeDtypeStruct((N_GRAPHS, N_PRED), jnp.float32),
    )(s, y, dinv, b, batch, W_lin, b_lin)


def kernel(x, edge_index, batch, W0, b0, W1, b1, W2, b2, W_lin, b_lin):
    srcf = edge_index[0]
    dstf = edge_index[1]
    dst_deg = dstf.reshape(NW, CH, K)
    bsrc, bdst, cnts = _bucket_call(srcf, dstf)
    bsrc4 = bsrc.reshape(NW, QP, MAXCH2, KA2)
    bdst4 = bdst.reshape(NW, QP, MAXCH2, KA2)
    deg2 = _deg_call(dst_deg)
    degs = deg2.reshape(NC, N2)[:, :N].reshape(NC, N, 1)
    y0, dinv = _tc_first(degs, x, W0)
    s0 = _agg_call(y0, bsrc4, bdst4, cnts)
    y1 = _tc_mid(s0, y0, dinv, b0.reshape(1, H), W1)
    s1 = _agg_call(y1, bsrc4, bdst4, cnts)
    y2 = _tc_mid(s1, y1, dinv, b1.reshape(1, H), W2)
    s2 = _agg_call(y2, bsrc4, bdst4, cnts)
    return _tc_final(s2, y2, dinv, b2.reshape(1, H), batch.reshape(N, 1),
                     W_lin, b_lin.reshape(1, N_PRED))
